# baseline (speedup 1.0000x reference)
import functools
import math

import jax
import jax.numpy as jnp
from jax import lax
from jax.experimental import pallas as pl
from jax.experimental.pallas import tpu as pltpu
from jax.experimental.pallas import tpu_sc as plsc

_VOCAB = 1000000
_EMB = 64
_B = 4096
_L = 200

_NC = 2
_NS = 16
_NW = _NC * _NS
_CB = _B // _NW
_ROWS = _VOCAB // 2
_SCALE = math.sqrt(float(_EMB))


def _make_sc_kernel():
    mesh = plsc.VectorSubcoreMesh(core_axis_name="c", subcore_axis_name="s")

    @functools.partial(
        pl.kernel,
        mesh=mesh,
        out_type=jax.ShapeDtypeStruct((_L, _EMB, _B), jnp.float32),
        scratch_types=[
            pltpu.VMEM((_L, _CB), jnp.int32),
            pltpu.VMEM((_CB, 128), jnp.float32),
            pltpu.VMEM((_CB, 128), jnp.float32),
            pltpu.VMEM((_EMB, _CB), jnp.float32),
            pltpu.VMEM((_CB,), jnp.int32),
            pltpu.VMEM((_CB,), jnp.int32),
            pltpu.SemaphoreType.DMA,
            pltpu.SemaphoreType.DMA,
        ],
        compiler_params=pltpu.CompilerParams(needs_layout_passes=False),
    )
    def sc_embed(tok_hbm, lt_hbm, out_hbm,
                 tok_v, gbuf0, gbuf1, obuf, idx0, idx1, s0, s1):
        wid = lax.axis_index("s") * _NC + lax.axis_index("c")
        band = wid * _CB

        pltpu.sync_copy(tok_hbm.at[:, pl.ds(band, _CB)], tok_v)

        gbufs = (gbuf0, gbuf1)
        idxs = (idx0, idx1)
        sems = (s0, s1)
        lanes = lax.iota(jnp.int32, 16)

        def fill_idx(l, b):
            for k in range(_CB // 16):
                sl = pl.ds(16 * k, 16)
                idxs[b][sl] = lax.shift_right_logical(tok_v[l, sl], 1)

        def start_gather(b):
            pltpu.async_copy(lt_hbm.at[idxs[b]], gbufs[b], sems[b])

        def wait_gather(b):
            pltpu.make_async_copy(lt_hbm.at[idxs[b]], gbufs[b], sems[b]).wait()

        def transpose_scale(l, b):
            g = gbufs[b]
            h64 = tuple(
                lax.shift_left(
                    lax.bitwise_and(tok_v[l, pl.ds(16 * k, 16)], 1), 6)
                for k in range(_CB // 16)
            )
            rows = tuple(lanes + 16 * k for k in range(_CB // 16))

            def e_body(e, _):
                ev = jnp.full((16,), e, jnp.int32)
                for k in range(_CB // 16):
                    col = h64[k] + ev
                    v = plsc.load_gather(g, [rows[k], col])
                    obuf[e, pl.ds(16 * k, 16)] = v * _SCALE
                return 0

            lax.fori_loop(0, _EMB, e_body, 0)

        def write_out(l):
            pltpu.sync_copy(obuf, out_hbm.at[l, :, pl.ds(band, _CB)])

        fill_idx(0, 0)
        start_gather(0)
        fill_idx(1, 1)
        start_gather(1)

        def body(i, _):
            for b in range(2):
                l = 2 * i + b
                wait_gather(b)
                transpose_scale(l, b)
                fill_idx(l + 2, b)
                start_gather(b)
                write_out(l)
            return 0

        lax.fori_loop(0, _L // 2 - 1, body, 0)

        for b in range(2):
            l = _L - 2 + b
            wait_gather(b)
            transpose_scale(l, b)
            write_out(l)

    return sc_embed


_sc_embed = _make_sc_kernel()


def kernel(tokens, table):
    tok_t = tokens.T
    lt = table.reshape(_ROWS, 128)
    out = _sc_embed(tok_t, lt)
    return out.transpose(2, 0, 1)


# speedup vs baseline: 1.4837x; 1.4837x over previous
import functools
import math

import jax
import jax.numpy as jnp
from jax import lax
from jax.experimental import pallas as pl
from jax.experimental.pallas import tpu as pltpu
from jax.experimental.pallas import tpu_sc as plsc

_VOCAB = 1000000
_EMB = 64
_B = 4096
_L = 200

_NC = 2
_NS = 16
_NW = _NC * _NS
_CB = _B // _NW
_NK = _CB // 16
_ROWS = _VOCAB // 2
_SCALE = math.sqrt(float(_EMB))


def _make_sc_kernel():
    mesh = plsc.VectorSubcoreMesh(core_axis_name="c", subcore_axis_name="s")

    @functools.partial(
        pl.kernel,
        mesh=mesh,
        out_type=jax.ShapeDtypeStruct((_L, _EMB, _B), jnp.float32),
        scratch_types=[
            pltpu.VMEM((_L, _CB), jnp.int32),
            pltpu.VMEM((_CB, 128), jnp.float32),
            pltpu.VMEM((_CB, 128), jnp.float32),
            pltpu.VMEM((_EMB, _CB), jnp.float32),
            pltpu.VMEM((_EMB, _CB), jnp.float32),
            pltpu.VMEM((_CB,), jnp.int32),
            pltpu.VMEM((_CB,), jnp.int32),
            pltpu.SemaphoreType.DMA,
            pltpu.SemaphoreType.DMA,
            pltpu.SemaphoreType.DMA,
            pltpu.SemaphoreType.DMA,
        ],
        compiler_params=pltpu.CompilerParams(needs_layout_passes=False),
    )
    def sc_embed(tok_hbm, lt_hbm, out_hbm,
                 tok_v, gbuf0, gbuf1, obuf0, obuf1, idx0, idx1,
                 g0, g1, w0, w1):
        wid = lax.axis_index("s") * _NC + lax.axis_index("c")
        band = wid * _CB

        pltpu.sync_copy(tok_hbm.at[:, pl.ds(band, _CB)], tok_v)

        gbufs = (gbuf0, gbuf1)
        obufs = (obuf0, obuf1)
        idxs = (idx0, idx1)
        gsems = (g0, g1)
        wsems = (w0, w1)
        lanes = lax.iota(jnp.int32, 16)
        rows = tuple(lanes + 16 * k for k in range(_NK))

        def fill_idx(l, b):
            for k in range(_NK):
                sl = pl.ds(16 * k, 16)
                idxs[b][sl] = lax.shift_right_logical(tok_v[l, sl], 1)

        def start_gather(b):
            pltpu.async_copy(lt_hbm.at[idxs[b]], gbufs[b], gsems[b])

        def wait_gather(b):
            pltpu.make_async_copy(
                lt_hbm.at[idxs[b]], gbufs[b], gsems[b]).wait()

        def transpose_scale(l, b):
            g = gbufs[b]
            o = obufs[b]

            def eg_body(eg, _):
                e0 = eg * 16
                for k in range(_NK):
                    sl = pl.ds(16 * k, 16)
                    he = lax.shift_left(
                        lax.bitwise_and(tok_v[l, sl], 1), 6) + e0
                    vs = [
                        plsc.load_gather(g, [rows[k], he + ei])
                        for ei in range(16)
                    ]
                    for ei in range(16):
                        o[e0 + ei, sl] = vs[ei] * _SCALE
                return 0

            lax.fori_loop(0, _EMB // 16, eg_body, 0)

        def start_write(l, b):
            pltpu.async_copy(
                obufs[b], out_hbm.at[l, :, pl.ds(band, _CB)], wsems[b])

        def wait_write(l, b):
            pltpu.make_async_copy(
                obufs[b], out_hbm.at[l, :, pl.ds(band, _CB)], wsems[b]).wait()

        fill_idx(0, 0)
        start_gather(0)
        fill_idx(1, 1)
        start_gather(1)

        def body(i, _):
            for b in range(2):
                l = 2 * i + b
                wait_gather(b)

                @pl.when(i >= 1)
                def _():
                    wait_write(l - 2, b)

                transpose_scale(l, b)

                @pl.when(i < _L // 2 - 1)
                def _():
                    fill_idx(l + 2, b)
                    start_gather(b)

                start_write(l, b)
            return 0

        lax.fori_loop(0, _L // 2, body, 0)

        for b in range(2):
            wait_write(_L - 2 + b, b)

    return sc_embed


_sc_embed = _make_sc_kernel()


def kernel(tokens, table):
    tok_t = tokens.T
    lt = table.reshape(_ROWS, 128)
    out = _sc_embed(tok_t, lt)
    return out.transpose(2, 0, 1)
